# baseline (speedup 1.0000x reference)
import functools

import jax
import jax.numpy as jnp
from jax import lax
from jax.experimental import pallas as pl
from jax.experimental.pallas import tpu as pltpu
from jax.experimental.pallas import tpu_sc as plsc

_L = 16
_NC = 2
_NS = 16
_NW = _NC * _NS
_EPS = 1e-6


def _build_sc_kernel(N, D, S, B, NCHUNK, CH):
    TPW = NCHUNK * CH
    SCALE = float(D) ** 0.5
    mesh = plsc.VectorSubcoreMesh(
        core_axis_name="c", subcore_axis_name="s",
        num_cores=_NC, num_subcores=_NS)

    @functools.partial(
        pl.kernel,
        out_type=jax.ShapeDtypeStruct((N, D), jnp.float32),
        mesh=mesh,
        scratch_types=[
            pltpu.VMEM((NCHUNK, CH), jnp.int32),
            pltpu.VMEM((B,), jnp.int32),
            pltpu.VMEM((B,), jnp.int32),
            pltpu.VMEM((_L,), jnp.float32),
            pltpu.VMEM((_L,), jnp.float32),
            pltpu.VMEM((B, D), jnp.float32),
            pltpu.VMEM((B, D), jnp.float32),
            pltpu.VMEM((D,), jnp.float32),
            pltpu.VMEM((D,), jnp.float32),
            pltpu.VMEM((CH, D), jnp.float32),
            pltpu.VMEM((CH, D), jnp.float32),
            pltpu.SemaphoreType.DMA,
            pltpu.SemaphoreType.DMA,
            pltpu.SemaphoreType.DMA,
            pltpu.SemaphoreType.DMA,
        ],
    )
    def k(tok_hbm, eid_hbm, sid_hbm, ge_hbm, gs_hbm, emo_hbm, scn_hbm,
          tab_hbm, w_hbm, out_hbm,
          tokidx_v, eid_v, sid_v, ge_v, gs_v, erows_v, srows_v,
          ctx_v, w_v, rows0_v, rows1_v,
          g0_sem, g1_sem, s0_sem, s1_sem):
        wid = lax.axis_index("s") * _NC + lax.axis_index("c")
        base = wid * TPW
        b = base // S
        bufs = (rows0_v, rows1_v)
        gsems = (g0_sem, g1_sem)
        ssems = (s0_sem, s1_sem)

        c_tok = pltpu.async_copy(tok_hbm.at[wid], tokidx_v, s0_sem)
        c_eid = pltpu.async_copy(eid_hbm, eid_v, s1_sem)
        c_sid = pltpu.async_copy(sid_hbm, sid_v, s1_sem)
        c_ge = pltpu.async_copy(ge_hbm, ge_v.at[pl.ds(0, 1)], s0_sem)
        c_gs = pltpu.async_copy(gs_hbm, gs_v.at[pl.ds(0, 1)], s0_sem)
        c_w = pltpu.async_copy(w_hbm, w_v, s1_sem)
        c_eid.wait()
        c_sid.wait()
        c_emo = pltpu.async_copy(emo_hbm.at[eid_v], erows_v, g0_sem)
        c_scn = pltpu.async_copy(scn_hbm.at[sid_v], srows_v, g1_sem)
        c_tok.wait()
        c_ge.wait()
        c_gs.wait()
        c_w.wait()
        c_emo.wait()
        c_scn.wait()

        lanes = lax.iota(jnp.int32, _L)
        dn = lax.GatherDimensionNumbers(
            offset_dims=(), collapsed_slice_dims=(0,), start_index_map=(0,))

        zero_idx = jnp.zeros((_L,), jnp.int32)
        inv_s = 1.0 / (float(D) ** 0.5)
        gev = lax.gather(ge_v[...], zero_idx[:, None], dn, slice_sizes=(1,),
                         mode=lax.GatherScatterMode.PROMISE_IN_BOUNDS) * inv_s
        gsv = lax.gather(gs_v[...], zero_idx[:, None], dn, slice_sizes=(1,),
                         mode=lax.GatherScatterMode.PROMISE_IN_BOUNDS) * inv_s

        def mkctx(j, carry):
            off = j * _L
            ctx_v[pl.ds(off, _L)] = (gev * erows_v[b, pl.ds(off, _L)]
                                     + gsv * srows_v[b, pl.ds(off, _L)])
            return carry
        lax.fori_loop(0, D // _L, mkctx, 0, unroll=4)

        RB = 4
        JU = 1

        def rms_of(accs):
            rvs = []
            for acc in accs:
                for sh in (1, 2, 4, 8):
                    g = lax.gather(
                        acc, (lanes ^ sh)[:, None], dn, slice_sizes=(1,),
                        mode=lax.GatherScatterMode.PROMISE_IN_BOUNDS)
                    acc = acc + g
                mv = acc * (1.0 / D) + (_EPS / D)
                iv = lax.bitcast_convert_type(mv, jnp.int32)
                iv = jnp.full((_L,), 0x5F3759DF, jnp.int32) - (iv >> 1)
                rv = lax.bitcast_convert_type(iv, jnp.float32)
                for _ in range(2):
                    rv = rv * (1.5 - 0.5 * mv * rv * rv)
                rvs.append(rv)
            return rvs

        def compute_chunk(rows_v):
            def p1_loop(r):
                def body(j2, accs):
                    new = list(accs)
                    for u in range(JU):
                        off = j2 * (JU * _L) + u * _L
                        cv = ctx_v[pl.ds(off, _L)]
                        for t in range(RB):
                            zv = rows_v[r + t, pl.ds(off, _L)] + cv
                            rows_v[r + t, pl.ds(off, _L)] = zv
                            new[t] = new[t] + zv * zv
                    return tuple(new)
                return plsc.parallel_loop(
                    0, D // (JU * _L),
                    carry=tuple(jnp.zeros((_L,), jnp.float32)
                                for _ in range(RB)))(body)

            def p2_loop(r, rvs):
                def body(j2):
                    for u in range(JU):
                        off = j2 * (JU * _L) + u * _L
                        wv = w_v[pl.ds(off, _L)]
                        for t in range(RB):
                            rows_v[r + t, pl.ds(off, _L)] = (
                                rows_v[r + t, pl.ds(off, _L)] * rvs[t] * wv)
                plsc.parallel_loop(0, D // (JU * _L))(body)

            def rowq_body(rq, carry):
                r = rq * RB
                p2_loop(r, rms_of(p1_loop(r)))
                return carry
            lax.fori_loop(0, CH // RB, rowq_body, 0)

        scatters = [None, None]
        gathers = [None, None]
        gathers[0] = pltpu.async_copy(
            tab_hbm.at[tokidx_v.at[0]], bufs[0], gsems[0])
        for c in range(NCHUNK):
            cur = c % 2
            nxt = (c + 1) % 2
            if c + 1 < NCHUNK:
                if scatters[nxt] is not None:
                    scatters[nxt].wait()
                    scatters[nxt] = None
                gathers[nxt] = pltpu.async_copy(
                    tab_hbm.at[tokidx_v.at[c + 1]], bufs[nxt], gsems[nxt])
            gathers[cur].wait()
            compute_chunk(bufs[cur])
            scatters[cur] = pltpu.async_copy(
                bufs[cur], out_hbm.at[pl.ds(base + c * CH, CH)], ssems[cur])
        for s in scatters:
            if s is not None:
                s.wait()

    return k


def kernel(token_ids, emotion_id, scene_id, token_table, emo_table,
           scene_table, gate_e, gate_s, norm_weight):
    B, S = token_ids.shape
    V, D = token_table.shape
    NE = emo_table.shape[0]
    N = B * S
    TPW = N // _NW
    CH = 32
    NCHUNK = TPW // CH

    tok3 = token_ids.reshape(_NW, NCHUNK, CH)
    ge1 = jnp.reshape(gate_e, (1,))
    gs1 = jnp.reshape(gate_s, (1,))

    k = _build_sc_kernel(N, D, S, B, NCHUNK, CH)
    out = k(tok3, emotion_id, scene_id, ge1, gs1, emo_table, scene_table,
            token_table, norm_weight)
    return out.reshape(B, S, D)


# speedup vs baseline: 1.2327x; 1.2327x over previous
import functools

import jax
import jax.numpy as jnp
from jax import lax
from jax.experimental import pallas as pl
from jax.experimental.pallas import tpu as pltpu
from jax.experimental.pallas import tpu_sc as plsc

_L = 16
_NC = 2
_NS = 16
_NW = _NC * _NS
_EPS = 1e-6


def _build_sc_kernel(N, D, S, B, NCHUNK, CH):
    TPW = NCHUNK * CH
    SCALE = float(D) ** 0.5
    mesh = plsc.VectorSubcoreMesh(
        core_axis_name="c", subcore_axis_name="s",
        num_cores=_NC, num_subcores=_NS)

    @functools.partial(
        pl.kernel,
        out_type=jax.ShapeDtypeStruct((N, D), jnp.float32),
        mesh=mesh,
        scratch_types=[
            pltpu.VMEM((NCHUNK, CH), jnp.int32),
            pltpu.VMEM((B,), jnp.int32),
            pltpu.VMEM((B,), jnp.int32),
            pltpu.VMEM((_L,), jnp.float32),
            pltpu.VMEM((_L,), jnp.float32),
            pltpu.VMEM((B, D), jnp.float32),
            pltpu.VMEM((B, D), jnp.float32),
            pltpu.VMEM((D,), jnp.float32),
            pltpu.VMEM((D,), jnp.float32),
            pltpu.VMEM((CH, D), jnp.float32),
            pltpu.VMEM((CH, D), jnp.float32),
            pltpu.SemaphoreType.DMA,
            pltpu.SemaphoreType.DMA,
            pltpu.SemaphoreType.DMA,
            pltpu.SemaphoreType.DMA,
        ],
    )
    def k(tok_hbm, eid_hbm, sid_hbm, ge_hbm, gs_hbm, emo_hbm, scn_hbm,
          tab_hbm, w_hbm, out_hbm,
          tokidx_v, eid_v, sid_v, ge_v, gs_v, erows_v, srows_v,
          ctx_v, w_v, rows0_v, rows1_v,
          g0_sem, g1_sem, s0_sem, s1_sem):
        wid = lax.axis_index("s") * _NC + lax.axis_index("c")
        base = wid * TPW
        b = base // S
        bufs = (rows0_v, rows1_v)
        gsems = (g0_sem, g1_sem)
        ssems = (s0_sem, s1_sem)

        c_tok = pltpu.async_copy(tok_hbm.at[wid], tokidx_v, s0_sem)
        c_eid = pltpu.async_copy(eid_hbm, eid_v, s1_sem)
        c_sid = pltpu.async_copy(sid_hbm, sid_v, s1_sem)
        c_ge = pltpu.async_copy(ge_hbm, ge_v.at[pl.ds(0, 1)], s0_sem)
        c_gs = pltpu.async_copy(gs_hbm, gs_v.at[pl.ds(0, 1)], s0_sem)
        c_w = pltpu.async_copy(w_hbm, w_v, s1_sem)
        c_eid.wait()
        c_sid.wait()
        c_emo = pltpu.async_copy(emo_hbm.at[eid_v], erows_v, g0_sem)
        c_scn = pltpu.async_copy(scn_hbm.at[sid_v], srows_v, g1_sem)
        c_tok.wait()
        c_ge.wait()
        c_gs.wait()
        c_w.wait()
        c_emo.wait()
        c_scn.wait()

        lanes = lax.iota(jnp.int32, _L)
        dn = lax.GatherDimensionNumbers(
            offset_dims=(), collapsed_slice_dims=(0,), start_index_map=(0,))

        zero_idx = jnp.zeros((_L,), jnp.int32)
        inv_s = 1.0 / (float(D) ** 0.5)
        gev = lax.gather(ge_v[...], zero_idx[:, None], dn, slice_sizes=(1,),
                         mode=lax.GatherScatterMode.PROMISE_IN_BOUNDS) * inv_s
        gsv = lax.gather(gs_v[...], zero_idx[:, None], dn, slice_sizes=(1,),
                         mode=lax.GatherScatterMode.PROMISE_IN_BOUNDS) * inv_s

        def mkctx(j, carry):
            off = j * _L
            ctx_v[pl.ds(off, _L)] = (gev * erows_v[b, pl.ds(off, _L)]
                                     + gsv * srows_v[b, pl.ds(off, _L)])
            return carry
        lax.fori_loop(0, D // _L, mkctx, 0, unroll=4)

        RB = 8
        JU = 1

        def rms_of(accs):
            rvs = []
            for acc in accs:
                for sh in (1, 2, 4, 8):
                    g = lax.gather(
                        acc, (lanes ^ sh)[:, None], dn, slice_sizes=(1,),
                        mode=lax.GatherScatterMode.PROMISE_IN_BOUNDS)
                    acc = acc + g
                mv = acc * (1.0 / D) + (_EPS / D)
                iv = lax.bitcast_convert_type(mv, jnp.int32)
                iv = jnp.full((_L,), 0x5F3759DF, jnp.int32) - (iv >> 1)
                rv = lax.bitcast_convert_type(iv, jnp.float32)
                for _ in range(2):
                    rv = rv * (1.5 - 0.5 * mv * rv * rv)
                rvs.append(rv)
            return rvs

        def compute_chunk(rows_v):
            def p1_loop(r):
                def body(j2, accs):
                    new = list(accs)
                    for u in range(JU):
                        off = j2 * (JU * _L) + u * _L
                        cv = ctx_v[pl.ds(off, _L)]
                        for t in range(RB):
                            zv = rows_v[r + t, pl.ds(off, _L)] + cv
                            rows_v[r + t, pl.ds(off, _L)] = zv
                            new[t] = new[t] + zv * zv
                    return tuple(new)
                return plsc.parallel_loop(
                    0, D // (JU * _L),
                    carry=tuple(jnp.zeros((_L,), jnp.float32)
                                for _ in range(RB)))(body)

            def p2_loop(r, rvs):
                def body(j2):
                    for u in range(JU):
                        off = j2 * (JU * _L) + u * _L
                        wv = w_v[pl.ds(off, _L)]
                        for t in range(RB):
                            rows_v[r + t, pl.ds(off, _L)] = (
                                rows_v[r + t, pl.ds(off, _L)] * rvs[t] * wv)
                plsc.parallel_loop(0, D // (JU * _L))(body)

            def rowq_body(rq, carry):
                r = rq * RB
                p2_loop(r, rms_of(p1_loop(r)))
                return carry
            lax.fori_loop(0, CH // RB, rowq_body, 0)

        scatters = [None, None]
        gathers = [None, None]
        gathers[0] = pltpu.async_copy(
            tab_hbm.at[tokidx_v.at[0]], bufs[0], gsems[0])
        for c in range(NCHUNK):
            cur = c % 2
            nxt = (c + 1) % 2
            if c + 1 < NCHUNK:
                if scatters[nxt] is not None:
                    scatters[nxt].wait()
                    scatters[nxt] = None
                gathers[nxt] = pltpu.async_copy(
                    tab_hbm.at[tokidx_v.at[c + 1]], bufs[nxt], gsems[nxt])
            gathers[cur].wait()
            compute_chunk(bufs[cur])
            scatters[cur] = pltpu.async_copy(
                bufs[cur], out_hbm.at[pl.ds(base + c * CH, CH)], ssems[cur])
        for s in scatters:
            if s is not None:
                s.wait()

    return k


def kernel(token_ids, emotion_id, scene_id, token_table, emo_table,
           scene_table, gate_e, gate_s, norm_weight):
    B, S = token_ids.shape
    V, D = token_table.shape
    NE = emo_table.shape[0]
    N = B * S
    TPW = N // _NW
    CH = 32
    NCHUNK = TPW // CH

    tok3 = token_ids.reshape(_NW, NCHUNK, CH)
    ge1 = jnp.reshape(gate_e, (1,))
    gs1 = jnp.reshape(gate_s, (1,))

    k = _build_sc_kernel(N, D, S, B, NCHUNK, CH)
    out = k(tok3, emotion_id, scene_id, ge1, gs1, emo_table, scene_table,
            token_table, norm_weight)
    return out.reshape(B, S, D)
